# baseline (speedup 1.0000x reference)
import functools

import jax
import jax.numpy as jnp
from jax import lax
from jax.experimental import pallas as pl
from jax.experimental.pallas import tpu as pltpu
from jax.experimental.pallas import tpu_sc as plsc

L = 16
NW = 32
A, B, Cdim = 1024, 64, 16
NP = B * Cdim
PPW = NP // NW
AH = A // 2
NCH = PPW * 2
NG = AH // L

_PAD_VREGS = (8, 12, 16, 20, 24, 28)


def _vgather(x, idx):
    dnums = lax.GatherDimensionNumbers(
        offset_dims=(), collapsed_slice_dims=(0,), start_index_map=(0,))
    return lax.gather(x, idx[:, None], dnums, slice_sizes=(1,),
                      mode=lax.GatherScatterMode.PROMISE_IN_BOUNDS)


def _rsqrt_nr(s):
    y = lax.bitcast_convert_type(
        jnp.int32(0x5F3759DF) - lax.shift_right_logical(
            lax.bitcast_convert_type(s, jnp.int32), 1),
        jnp.float32,
    )
    for _ in range(4):
        y = y * (1.5 - 0.5 * s * y * y)
    return y


_mesh = plsc.VectorSubcoreMesh(core_axis_name="c", subcore_axis_name="s")


@functools.partial(
    pl.kernel,
    mesh=_mesh,
    compiler_params=pltpu.CompilerParams(
        needs_layout_passes=False, use_tc_tiling_on_sc=False),
    out_type=jax.ShapeDtypeStruct((B, Cdim, 64, A), jnp.float32),
    scratch_types=[
        pltpu.VMEM((512,), jnp.float32),
        pltpu.VMEM((2048,), jnp.float32),
        pltpu.VMEM((8, AH), jnp.int32),
        pltpu.VMEM((8, AH), jnp.int32),
        pltpu.VMEM((64, AH), jnp.float32),
        pltpu.VMEM((64, AH), jnp.float32),
        pltpu.SemaphoreType.DMA,
        pltpu.SemaphoreType.DMA,
        pltpu.SemaphoreType.DMA,
        pltpu.SemaphoreType.DMA,
    ],
)
def _embed_sc(x_hbm, t0, t1, t2, t3, t4, t5, t6, t7, out_hbm, raw_v, lut_v,
              x_v0, x_v1, o_v0, o_v1, in_s0, in_s1, out_s0, out_s1):
    wid = lax.axis_index("s") * 2 + lax.axis_index("c")
    iota = lax.iota(jnp.int32, L)

    for k, t in enumerate((t0, t1, t2, t3, t4, t5, t6, t7)):
        pltpu.sync_copy(t, raw_v.at[pl.ds(k * 64, 64)])

    padmask = jnp.where(iota < 8, 0.0, 1.0)
    lane8 = iota & 7
    hi = iota >> 3
    lutbase = lane8 * 256 + hi * 4
    for r in range(32):
        v = raw_v[pl.ds(r * L, L)]
        if r in _PAD_VREGS:
            v = v * padmask
        s = v * v
        for sh in (1, 2, 4):
            s = s + _vgather(s, iota ^ sh)
        s = jnp.maximum(s, 1e-24)
        scale = jnp.minimum(_rsqrt_nr(s), 1.0)
        sv = v * scale
        coff = (r // 4) * 32 + ((2 * r) % 8) * 4
        plsc.store_scatter(lut_v, [lutbase + coff], sv)
        plsc.store_scatter(lut_v, [lutbase + (coff + 1)], sv)
        plsc.store_scatter(lut_v, [lutbase + (coff + 2)], sv)
        plsc.store_scatter(lut_v, [lutbase + (coff + 3)], sv)

    in_sems = (in_s0, in_s1)
    out_sems = (out_s0, out_s1)
    x_bufs = (x_v0, x_v1)
    o_bufs = (o_v0, o_v1)
    pbase = wid * PPW

    def x_copy(cg, b):
        p = pbase + cg
        return pltpu.make_async_copy(
            x_hbm.at[lax.div(p, Cdim), lax.rem(p, Cdim), :,
                     pl.ds(b * AH, AH)],
            x_bufs[b], in_sems[b])

    def o_copy(cg, b):
        p = pbase + cg
        return pltpu.make_async_copy(
            o_bufs[b],
            out_hbm.at[lax.div(p, Cdim), lax.rem(p, Cdim), :,
                       pl.ds(b * AH, AH)],
            out_sems[b])

    x_copy(0, 0).start()

    def chunk_group(cg, _):
        for b in range(2):
            ci = 2 * cg + b
            x_copy(cg, b).wait()

            @pl.when(ci + 1 < NCH)
            def _():
                x_copy(cg + b, 1 - b).start()

            @pl.when(ci >= 2)
            def _():
                o_copy(cg - 1, b).wait()

            ob = o_bufs[b]
            xb = x_bufs[b]

            q4 = iota & 3

            @plsc.parallel_loop(0, NG, unroll=2)
            def g_body(g):
                bases = [xb[i, pl.ds(g * L, L)] * 4 + q4 for i in range(8)]
                for d in range(64):
                    idx = bases[d // 8] + ((d % 8) * 8 + d // 8) * 32
                    ob[d, pl.ds(g * L, L)] = plsc.load_gather(lut_v, [idx])

            o_copy(cg, b).start()
        return 0

    lax.fori_loop(0, NCH // 2, chunk_group, 0)
    o_copy(PPW - 1, 0).wait()
    o_copy(PPW - 1, 1).wait()


def kernel(x, octave_w, pitch_w, short_dur_w, medium_dur_w, long_dur_w,
           velocity_w, short_shift_w, long_shift_w):
    tabs = [w[:8].reshape(-1) for w in (octave_w, pitch_w, short_dur_w,
                                        medium_dur_w, long_dur_w, velocity_w,
                                        short_shift_w, long_shift_w)]
    x_t = jnp.transpose(x.astype(jnp.int32), (1, 2, 3, 0))
    out_t = _embed_sc(x_t, *tabs)
    return jnp.transpose(out_t, (3, 0, 1, 2))


# speedup vs baseline: 1.0378x; 1.0378x over previous
import functools

import jax
import jax.numpy as jnp
from jax import lax
from jax.experimental import pallas as pl
from jax.experimental.pallas import tpu as pltpu
from jax.experimental.pallas import tpu_sc as plsc

L = 16
NW = 32
A, B, Cdim = 1024, 64, 16
NP = B * Cdim
PPW = NP // NW
AH = A // 2
NCH = PPW * 2
NG = AH // L

_PAD_VREGS = (8, 12, 16, 20, 24, 28)


def _vgather(x, idx):
    dnums = lax.GatherDimensionNumbers(
        offset_dims=(), collapsed_slice_dims=(0,), start_index_map=(0,))
    return lax.gather(x, idx[:, None], dnums, slice_sizes=(1,),
                      mode=lax.GatherScatterMode.PROMISE_IN_BOUNDS)


def _rsqrt_nr(s):
    y = lax.bitcast_convert_type(
        jnp.int32(0x5F3759DF) - lax.shift_right_logical(
            lax.bitcast_convert_type(s, jnp.int32), 1),
        jnp.float32,
    )
    for _ in range(4):
        y = y * (1.5 - 0.5 * s * y * y)
    return y


_mesh = plsc.VectorSubcoreMesh(core_axis_name="c", subcore_axis_name="s")


@functools.partial(
    pl.kernel,
    mesh=_mesh,
    compiler_params=pltpu.CompilerParams(
        needs_layout_passes=False, use_tc_tiling_on_sc=False),
    out_type=jax.ShapeDtypeStruct((B, Cdim, 64, A), jnp.float32),
    scratch_types=[
        pltpu.VMEM((512,), jnp.float32),
        pltpu.VMEM((1024,), jnp.float32),
        pltpu.VMEM((8, AH), jnp.int32),
        pltpu.VMEM((8, AH), jnp.int32),
        pltpu.VMEM((64, AH), jnp.float32),
        pltpu.VMEM((64, AH), jnp.float32),
        pltpu.SemaphoreType.DMA,
        pltpu.SemaphoreType.DMA,
        pltpu.SemaphoreType.DMA,
        pltpu.SemaphoreType.DMA,
    ],
)
def _embed_sc(x_hbm, t0, t1, t2, t3, t4, t5, t6, t7, out_hbm, raw_v, lut_v,
              x_v0, x_v1, o_v0, o_v1, in_s0, in_s1, out_s0, out_s1):
    wid = lax.axis_index("s") * 2 + lax.axis_index("c")
    iota = lax.iota(jnp.int32, L)

    for k, t in enumerate((t0, t1, t2, t3, t4, t5, t6, t7)):
        pltpu.sync_copy(t, raw_v.at[pl.ds(k * 64, 64)])

    padmask = jnp.where(iota < 8, 0.0, 1.0)
    lane8 = iota & 7
    hi = iota >> 3
    lutbase = lane8 * 128 + hi * 2
    for r in range(32):
        v = raw_v[pl.ds(r * L, L)]
        if r in _PAD_VREGS:
            v = v * padmask
        s = v * v
        for sh in (1, 2, 4):
            s = s + _vgather(s, iota ^ sh)
        s = jnp.maximum(s, 1e-24)
        scale = jnp.minimum(_rsqrt_nr(s), 1.0)
        sv = v * scale
        coff = (r // 4) * 16 + ((2 * r) % 8) * 2
        plsc.store_scatter(lut_v, [lutbase + coff], sv)
        plsc.store_scatter(lut_v, [lutbase + (coff + 1)], sv)

    in_sems = (in_s0, in_s1)
    out_sems = (out_s0, out_s1)
    x_bufs = (x_v0, x_v1)
    o_bufs = (o_v0, o_v1)
    pbase = wid * PPW

    def x_copy(cg, b):
        p = pbase + cg
        return pltpu.make_async_copy(
            x_hbm.at[lax.div(p, Cdim), lax.rem(p, Cdim), :,
                     pl.ds(b * AH, AH)],
            x_bufs[b], in_sems[b])

    def o_copy(cg, b):
        p = pbase + cg
        return pltpu.make_async_copy(
            o_bufs[b],
            out_hbm.at[lax.div(p, Cdim), lax.rem(p, Cdim), :,
                       pl.ds(b * AH, AH)],
            out_sems[b])

    x_copy(0, 0).start()

    def chunk_group(cg, _):
        for b in range(2):
            ci = 2 * cg + b
            x_copy(cg, b).wait()

            @pl.when(ci + 1 < NCH)
            def _():
                x_copy(cg + b, 1 - b).start()

            @pl.when(ci >= 2)
            def _():
                o_copy(cg - 1, b).wait()

            ob = o_bufs[b]
            xb = x_bufs[b]

            @plsc.parallel_loop(0, NG, unroll=2)
            def g_body(g):
                bases = [xb[i, pl.ds(g * L, L)] * 2 + hi for i in range(8)]
                for d in range(64):
                    idx = bases[d // 8] + ((d % 8) * 8 + d // 8) * 16
                    ob[d, pl.ds(g * L, L)] = plsc.load_gather(lut_v, [idx])

            o_copy(cg, b).start()
        return 0

    lax.fori_loop(0, NCH // 2, chunk_group, 0)
    o_copy(PPW - 1, 0).wait()
    o_copy(PPW - 1, 1).wait()


def kernel(x, octave_w, pitch_w, short_dur_w, medium_dur_w, long_dur_w,
           velocity_w, short_shift_w, long_shift_w):
    tabs = [w[:8].reshape(-1) for w in (octave_w, pitch_w, short_dur_w,
                                        medium_dur_w, long_dur_w, velocity_w,
                                        short_shift_w, long_shift_w)]
    x_t = jnp.transpose(x.astype(jnp.int32), (1, 2, 3, 0))
    out_t = _embed_sc(x_t, *tabs)
    return jnp.transpose(out_t, (3, 0, 1, 2))
